# baseline (speedup 1.0000x reference)
import functools

import jax
import jax.numpy as jnp
from jax import lax
from jax.experimental import pallas as pl
from jax.experimental.pallas import tpu as pltpu
from jax.experimental.pallas import tpu_sc as plsc

HID = 768
NE = 64
T = 32768
H2 = 2 * HID

NC = 2
NS = 16
L = 16
NW = NC * NS
CHUNK = T // NW
CORE_TOK = T // NC

BR = 512
BT = 256
CAPC = CORE_TOK + BT
NBLK = CORE_TOK // BT


def _gate_weight(logits):
  col = lax.broadcasted_iota(jnp.int32, logits.shape, 1)
  is_last = col == (NE - 1)
  neg = jnp.float32(-jnp.inf)
  l_last = jnp.max(jnp.where(is_last, logits, neg), axis=1, keepdims=True)
  m_others = jnp.max(jnp.where(is_last, neg, logits), axis=1, keepdims=True)
  cnt = jnp.sum(
      jnp.where(is_last | (logits < l_last), 0.0, 1.0), axis=1, keepdims=True
  )
  sel = cnt <= 1.0
  return jnp.where(sel, jax.nn.sigmoid(l_last - m_others), 0.0)


def _router_body(x_ref, wr_ref, br_ref, w_ref, cnt_ref):
  logits = (
      jnp.dot(x_ref[...], wr_ref[...], preferred_element_type=jnp.float32)
      + br_ref[...]
  )
  wgt = _gate_weight(logits)
  w_ref[...] = wgt
  cnt_ref[...] = jnp.sum((wgt > 0.0).astype(jnp.float32)).reshape(1, 1, 1)


def _router(x, wr, br2):
  return pl.pallas_call(
      _router_body,
      grid=(T // BR,),
      in_specs=[
          pl.BlockSpec((BR, HID), lambda i: (i, 0)),
          pl.BlockSpec((HID, NE), lambda i: (0, 0)),
          pl.BlockSpec((1, NE), lambda i: (0, 0)),
      ],
      out_specs=[
          pl.BlockSpec((BR, 1), lambda i: (i, 0)),
          pl.BlockSpec((1, 1, 1), lambda i: (i, 0, 0)),
      ],
      out_shape=[
          jax.ShapeDtypeStruct((T, 1), jnp.float32),
          jax.ShapeDtypeStruct((T // BR, 1, 1), jnp.float32),
      ],
  )(x, wr, br2)


def _compact_body(w_hbm, x_hbm, bases_hbm, xc, wv, lids, rows, bb, sem):
  cid = lax.axis_index("c")
  sid = lax.axis_index("s")
  gw = cid * NS + sid
  base_tok = gw * CHUNK
  pltpu.sync_copy(w_hbm.at[pl.ds(base_tok, CHUNK)], wv)
  pltpu.sync_copy(bases_hbm, bb)
  iota = lax.iota(jnp.int32, L)

  def step(j, off):
    v = wv[pl.ds(j * L, L)]
    m = v > 0.0
    ids = base_tok + j * L + iota
    pos = off + plsc.cumsum(jnp.where(m, 1, 0).astype(jnp.int32)) - 1
    plsc.store_scatter(lids, [pos], ids, mask=m)
    return off + plsc.all_reduce_population_count(m)[0]

  cnt = lax.fori_loop(0, CHUNK // L, step, jnp.int32(0))

  cnt_pad = ((cnt + (L - 1)) // L) * L
  plsc.store_scatter(
      lids, [cnt + iota], jnp.full((L,), base_tok, jnp.int32),
      mask=iota < (cnt_pad - cnt),
  )

  gbase = plsc.load_gather(bb, [jnp.full((L,), gw, jnp.int32)])[0]

  def scat(k, _):
    idv = lids[pl.ds(k * L, L)]
    off = pl.multiple_of(gbase + k * L, L)
    pltpu.async_copy(x_hbm.at[idv], rows, sem).wait()
    pltpu.sync_copy(rows, xc.at[pl.ds(off, L)])
    return 0

  lax.fori_loop(0, cnt_pad // L, scat, 0)


def _compact(w_flat, x, bases):
  mesh = plsc.VectorSubcoreMesh(core_axis_name="c", subcore_axis_name="s", num_cores=NC, num_subcores=NS)
  kern = pl.kernel(
      _compact_body,
      out_type=jax.ShapeDtypeStruct((NC * CAPC, HID), jnp.float32),
      mesh=mesh,
      scratch_types=[
          pltpu.VMEM((CHUNK,), jnp.float32),
          pltpu.VMEM((CHUNK + L,), jnp.int32),
          pltpu.VMEM((L, HID), jnp.float32),
          pltpu.VMEM((NW,), jnp.int32),
          pltpu.SemaphoreType.DMA,
      ],
      compiler_params=pltpu.CompilerParams(needs_layout_passes=False),
  )
  return kern(w_flat, x, bases)


def _mlp_body(cnt_ref, xc_ref, wr_ref, br_ref, w1_ref, b1_ref, w2_ref, b2_ref,
              yc_ref):
  r = pl.program_id(0)
  j = pl.program_id(1)
  cnt = cnt_ref[r]

  @pl.when(j * BT < cnt)
  def _():
    xb = xc_ref[...]
    logits = (
        jnp.dot(xb, wr_ref[...], preferred_element_type=jnp.float32)
        + br_ref[...]
    )
    wgt = _gate_weight(logits)
    h = jnp.maximum(
        jnp.dot(xb, w1_ref[...], preferred_element_type=jnp.float32)
        + b1_ref[...],
        0.0,
    )
    y = jnp.dot(h, w2_ref[...], preferred_element_type=jnp.float32) + b2_ref[...]
    yc_ref[...] = y * wgt


DUMP_BLK = (NC * CAPC) // BT


def _mlp(cnts, xc, wr, br2, w1, b12, w2, b22):
  def xmap(r, j, cnt_ref):
    nb = jnp.maximum(1, (cnt_ref[r] + BT - 1) // BT)
    jj = jnp.minimum(j, nb - 1)
    return (r * (CAPC // BT) + jj, 0)

  def omap(r, j, cnt_ref):
    nb = (cnt_ref[r] + BT - 1) // BT
    return (jnp.where(j < nb, r * (CAPC // BT) + jnp.minimum(j, nb - 1),
                      DUMP_BLK), 0)

  grid_spec = pltpu.PrefetchScalarGridSpec(
      num_scalar_prefetch=1,
      grid=(NC, NBLK),
      in_specs=[
          pl.BlockSpec((BT, HID), xmap),
          pl.BlockSpec((HID, NE), lambda r, j, c: (0, 0)),
          pl.BlockSpec((1, NE), lambda r, j, c: (0, 0)),
          pl.BlockSpec((HID, H2), lambda r, j, c: (0, 0)),
          pl.BlockSpec((1, H2), lambda r, j, c: (0, 0)),
          pl.BlockSpec((H2, HID), lambda r, j, c: (0, 0)),
          pl.BlockSpec((1, HID), lambda r, j, c: (0, 0)),
      ],
      out_specs=pl.BlockSpec((BT, HID), xmap),
  )
  return pl.pallas_call(
      _mlp_body,
      grid_spec=grid_spec,
      out_shape=jax.ShapeDtypeStruct((NC * CAPC + BT, HID), jnp.float32),
  )(cnts, xc, wr, br2, w1, b12, w2, b22)


ZROW = NC * CAPC


def _scatter_body(yc, w_hbm, bases_hbm, out, zbuf, mapb, wv, bb, rows, sem):
  cid = lax.axis_index("c")
  sid = lax.axis_index("s")
  gw = cid * NS + sid
  slab = gw * CHUNK
  iota = lax.iota(jnp.int32, L)

  pltpu.sync_copy(w_hbm.at[pl.ds(slab, CHUNK)], wv)
  pltpu.sync_copy(bases_hbm, bb)
  pltpu.sync_copy(yc.at[pl.ds(ZROW, L)], zbuf)

  gbase = plsc.load_gather(bb, [jnp.full((L,), gw, jnp.int32)])[0]

  def mstep(j, off):
    v = wv[pl.ds(j * L, L)]
    m = v > 0.0
    rank = off + plsc.cumsum(jnp.where(m, 1, 0).astype(jnp.int32)) - 1
    plsc.store_scatter(
        mapb, [j * L + iota], jnp.where(m, gbase + rank, ZROW))
    return off + plsc.all_reduce_population_count(m)[0]

  lax.fori_loop(0, CHUNK // L, mstep, jnp.int32(0))

  GB = 128

  def emit(g, _):
    off = pl.multiple_of(slab + g * GB, GB)
    pltpu.async_copy(yc.at[mapb.at[pl.ds(g * GB, GB)]], rows, sem).wait()
    pltpu.sync_copy(rows, out.at[pl.ds(off, GB)])
    return 0

  lax.fori_loop(0, CHUNK // GB, emit, 0)


def _scatter(yc, w_flat, bases):
  mesh = plsc.VectorSubcoreMesh(core_axis_name="c", subcore_axis_name="s", num_cores=NC, num_subcores=NS)
  kern = pl.kernel(
      _scatter_body,
      out_type=jax.ShapeDtypeStruct((T, HID), jnp.float32),
      mesh=mesh,
      scratch_types=[
          pltpu.VMEM((L, HID), jnp.float32),
          pltpu.VMEM((CHUNK,), jnp.int32),
          pltpu.VMEM((CHUNK,), jnp.float32),
          pltpu.VMEM((NW,), jnp.int32),
          pltpu.VMEM((128, HID), jnp.float32),
          pltpu.SemaphoreType.DMA,
      ],
      compiler_params=pltpu.CompilerParams(needs_layout_passes=False),
  )
  return kern(yc, w_flat, bases)


def kernel(x, Wr, br, W1, b1, W2, b2):
  br2 = br.reshape(1, NE)
  b12 = b1.reshape(1, H2)
  b22 = b2.reshape(1, HID)
  w, blkcnt = _router(x, Wr, br2)
  w_flat = w.reshape(T)
  tilecnt = blkcnt.reshape(NW, (T // BR) // NW).sum(axis=1).astype(jnp.int32)
  cpad = ((tilecnt + (L - 1)) // L) * L
  cp2 = cpad.reshape(NC, NS)
  ex = jnp.cumsum(cp2, axis=1) - cp2
  bases = (jnp.arange(NC, dtype=jnp.int32)[:, None] * CAPC + ex).reshape(NW)
  cnts = cp2.sum(axis=1)
  xc = _compact(w_flat, x, bases)
  yc = _mlp(cnts, xc, Wr, br2, W1, b12, W2, b22)
  return _scatter(yc, w_flat, bases)


# speedup vs baseline: 4.4251x; 4.4251x over previous
import functools

import jax
import jax.numpy as jnp
from jax import lax
from jax.experimental import pallas as pl
from jax.experimental.pallas import tpu as pltpu
from jax.experimental.pallas import tpu_sc as plsc

HID = 768
NE = 64
T = 32768
H2 = 2 * HID

NC = 2
NS = 16
L = 16
NW = NC * NS
CHUNK = T // NW
CORE_TOK = T // NC

BR = 512
BT = 256
CAPC = CORE_TOK + BT
NBLK = CORE_TOK // BT


def _gate_weight(logits):
  col = lax.broadcasted_iota(jnp.int32, logits.shape, 1)
  is_last = col == (NE - 1)
  neg = jnp.float32(-jnp.inf)
  l_last = jnp.max(jnp.where(is_last, logits, neg), axis=1, keepdims=True)
  m_others = jnp.max(jnp.where(is_last, neg, logits), axis=1, keepdims=True)
  cnt = jnp.sum(
      jnp.where(is_last | (logits < l_last), 0.0, 1.0), axis=1, keepdims=True
  )
  sel = cnt <= 1.0
  return jnp.where(sel, jax.nn.sigmoid(l_last - m_others), 0.0)


def _router_body(x_ref, wr_ref, br_ref, w_ref, cnt_ref):
  logits = (
      jnp.dot(x_ref[...], wr_ref[...], preferred_element_type=jnp.float32)
      + br_ref[...]
  )
  wgt = _gate_weight(logits)
  w_ref[...] = wgt
  cnt_ref[...] = jnp.sum((wgt > 0.0).astype(jnp.float32)).reshape(1, 1, 1)


def _router(x, wr, br2):
  return pl.pallas_call(
      _router_body,
      grid=(T // BR,),
      in_specs=[
          pl.BlockSpec((BR, HID), lambda i: (i, 0)),
          pl.BlockSpec((HID, NE), lambda i: (0, 0)),
          pl.BlockSpec((1, NE), lambda i: (0, 0)),
      ],
      out_specs=[
          pl.BlockSpec((BR, 1), lambda i: (i, 0)),
          pl.BlockSpec((1, 1, 1), lambda i: (i, 0, 0)),
      ],
      out_shape=[
          jax.ShapeDtypeStruct((T, 1), jnp.float32),
          jax.ShapeDtypeStruct((T // BR, 1, 1), jnp.float32),
      ],
  )(x, wr, br2)


def _compact_body(w_hbm, x_hbm, bases_hbm, xc, wv, lids, rows, bb, sem):
  cid = lax.axis_index("c")
  sid = lax.axis_index("s")
  gw = cid * NS + sid
  base_tok = gw * CHUNK
  pltpu.sync_copy(w_hbm.at[pl.ds(base_tok, CHUNK)], wv)
  pltpu.sync_copy(bases_hbm, bb)
  iota = lax.iota(jnp.int32, L)

  def step(j, carry):
    off, uid = carry
    v = wv[pl.ds(j * L, L)]
    m = v > 0.0
    ids = base_tok + j * L + iota
    pos = off + plsc.cumsum(jnp.where(m, 1, 0).astype(jnp.int32)) - 1
    plsc.store_scatter(lids, [pos], ids, mask=m)
    un = jnp.logical_not(m)
    has_un = plsc.all_reduce_population_count(un)[0] > 0
    f = plsc.all_reduce_ffs(un)[0]
    uid = jnp.where((uid < 0) & has_un, base_tok + j * L + f, uid)
    return (off + plsc.all_reduce_population_count(m)[0], uid)

  cnt, uid = lax.fori_loop(0, CHUNK // L, step,
                           (jnp.int32(0), jnp.int32(-1)))

  padid = jnp.where(uid < 0, base_tok, uid)
  cnt_pad = (cnt // L) * L + L
  plsc.store_scatter(
      lids, [cnt + iota], jnp.full((L,), 1, jnp.int32) * padid,
      mask=iota < (cnt_pad - cnt),
  )

  gbase = plsc.load_gather(bb, [jnp.full((L,), gw, jnp.int32)])[0]

  def scat(k, _):
    idv = lids[pl.ds(k * L, L)]
    off = pl.multiple_of(gbase + k * L, L)
    pltpu.async_copy(x_hbm.at[idv], rows, sem).wait()
    pltpu.sync_copy(rows, xc.at[pl.ds(off, L)])
    return 0

  lax.fori_loop(0, cnt_pad // L, scat, 0)


def _compact(w_flat, x, bases):
  mesh = plsc.VectorSubcoreMesh(core_axis_name="c", subcore_axis_name="s", num_cores=NC, num_subcores=NS)
  kern = pl.kernel(
      _compact_body,
      out_type=jax.ShapeDtypeStruct((NC * CAPC, HID), jnp.float32),
      mesh=mesh,
      scratch_types=[
          pltpu.VMEM((CHUNK,), jnp.float32),
          pltpu.VMEM((CHUNK + L,), jnp.int32),
          pltpu.VMEM((L, HID), jnp.float32),
          pltpu.VMEM((NW,), jnp.int32),
          pltpu.SemaphoreType.DMA,
      ],
      compiler_params=pltpu.CompilerParams(needs_layout_passes=False),
  )
  return kern(w_flat, x, bases)


def _mlp_body(cnt_ref, xc_ref, wr_ref, br_ref, w1_ref, b1_ref, w2_ref, b2_ref,
              yc_ref):
  r = pl.program_id(0)
  j = pl.program_id(1)
  cnt = cnt_ref[r]

  @pl.when(j * BT < cnt)
  def _():
    xb = xc_ref[...]
    logits = (
        jnp.dot(xb, wr_ref[...], preferred_element_type=jnp.float32)
        + br_ref[...]
    )
    wgt = _gate_weight(logits)
    h = jnp.maximum(
        jnp.dot(xb, w1_ref[...], preferred_element_type=jnp.float32)
        + b1_ref[...],
        0.0,
    )
    y = jnp.dot(h, w2_ref[...], preferred_element_type=jnp.float32) + b2_ref[...]
    yc_ref[...] = y * wgt


DUMP_BLK = (NC * CAPC) // BT


def _mlp(cnts, xc, wr, br2, w1, b12, w2, b22):
  def xmap(r, j, cnt_ref):
    nb = jnp.maximum(1, (cnt_ref[r] + BT - 1) // BT)
    jj = jnp.minimum(j, nb - 1)
    return (r * (CAPC // BT) + jj, 0)

  def omap(r, j, cnt_ref):
    nb = (cnt_ref[r] + BT - 1) // BT
    return (jnp.where(j < nb, r * (CAPC // BT) + jnp.minimum(j, nb - 1),
                      DUMP_BLK), 0)

  grid_spec = pltpu.PrefetchScalarGridSpec(
      num_scalar_prefetch=1,
      grid=(NC, NBLK),
      in_specs=[
          pl.BlockSpec((BT, HID), xmap),
          pl.BlockSpec((HID, NE), lambda r, j, c: (0, 0)),
          pl.BlockSpec((1, NE), lambda r, j, c: (0, 0)),
          pl.BlockSpec((HID, H2), lambda r, j, c: (0, 0)),
          pl.BlockSpec((1, H2), lambda r, j, c: (0, 0)),
          pl.BlockSpec((H2, HID), lambda r, j, c: (0, 0)),
          pl.BlockSpec((1, HID), lambda r, j, c: (0, 0)),
      ],
      out_specs=pl.BlockSpec((BT, HID), xmap),
  )
  return pl.pallas_call(
      _mlp_body,
      grid_spec=grid_spec,
      out_shape=jax.ShapeDtypeStruct((NC * CAPC + BT, HID), jnp.float32),
  )(cnts, xc, wr, br2, w1, b12, w2, b22)


ZROW = NC * CAPC


def _scatter_body(yc, w_hbm, bases_hbm, out, zbuf, mapb, wv, bb, rows, sem):
  cid = lax.axis_index("c")
  sid = lax.axis_index("s")
  gw = cid * NS + sid
  slab = gw * CHUNK
  iota = lax.iota(jnp.int32, L)

  pltpu.sync_copy(w_hbm.at[pl.ds(slab, CHUNK)], wv)
  pltpu.sync_copy(bases_hbm, bb)
  pltpu.sync_copy(yc.at[pl.ds(ZROW, L)], zbuf)

  gbase = plsc.load_gather(bb, [jnp.full((L,), gw, jnp.int32)])[0]
  GB = 128

  def cstep(j, off):
    v = wv[pl.ds(j * L, L)]
    return off + plsc.all_reduce_population_count(v > 0.0)[0]

  cnt = lax.fori_loop(0, CHUNK // L, cstep, jnp.int32(0))
  zbase = gbase + cnt
  npad = (cnt // L) * L + L - cnt

  def mstep(j, off):
    v = wv[pl.ds(j * L, L)]
    m = v > 0.0
    rank = off + plsc.cumsum(jnp.where(m, 1, 0).astype(jnp.int32)) - 1
    p = j * L + iota
    plsc.store_scatter(
        mapb, [p], jnp.where(m, gbase + rank, zbase + (p % npad)))
    return off + plsc.all_reduce_population_count(m)[0]

  lax.fori_loop(0, CHUNK // L, mstep, jnp.int32(0))

  def emit(g, _):
    off = pl.multiple_of(slab + g * GB, GB)
    pltpu.async_copy(yc.at[mapb.at[pl.ds(g * GB, GB)]], rows, sem).wait()
    pltpu.sync_copy(rows, out.at[pl.ds(off, GB)])
    return 0

  lax.fori_loop(0, CHUNK // GB, emit, 0)


def _scatter(yc, w_flat, bases):
  mesh = plsc.VectorSubcoreMesh(core_axis_name="c", subcore_axis_name="s", num_cores=NC, num_subcores=NS)
  kern = pl.kernel(
      _scatter_body,
      out_type=jax.ShapeDtypeStruct((T, HID), jnp.float32),
      mesh=mesh,
      scratch_types=[
          pltpu.VMEM((L, HID), jnp.float32),
          pltpu.VMEM((CHUNK,), jnp.int32),
          pltpu.VMEM((CHUNK,), jnp.float32),
          pltpu.VMEM((NW,), jnp.int32),
          pltpu.VMEM((128, HID), jnp.float32),
          pltpu.SemaphoreType.DMA,
      ],
      compiler_params=pltpu.CompilerParams(needs_layout_passes=False),
  )
  return kern(yc, w_flat, bases)


def kernel(x, Wr, br, W1, b1, W2, b2):
  br2 = br.reshape(1, NE)
  b12 = b1.reshape(1, H2)
  b22 = b2.reshape(1, HID)
  w, blkcnt = _router(x, Wr, br2)
  w_flat = w.reshape(T)
  tilecnt = blkcnt.reshape(NW, (T // BR) // NW).sum(axis=1).astype(jnp.int32)
  cpad = (tilecnt // L) * L + L
  cp2 = cpad.reshape(NC, NS)
  ex = jnp.cumsum(cp2, axis=1) - cp2
  bases = (jnp.arange(NC, dtype=jnp.int32)[:, None] * CAPC + ex).reshape(NW)
  cnts = cp2.sum(axis=1)
  xc = _compact(w_flat, x, bases)
  yc = _mlp(cnts, xc, Wr, br2, W1, b12, W2, b22)
  return _scatter(yc, w_flat, bases)
